# Initial kernel scaffold; baseline (speedup 1.0000x reference)
import jax
import jax.numpy as jnp
from jax.experimental import pallas as pl


def kernel(x, edge_index, W, b):
    raise NotImplementedError("write your pallas kernel here")


# speedup vs baseline: 107.7212x; 107.7212x over previous
import functools

import jax
import jax.numpy as jnp
from jax import lax
from jax.experimental import pallas as pl
from jax.experimental.pallas import tpu as pltpu
from jax.experimental.pallas import tpu_sc as plsc

_NC = 2
_NS = 16
_NW = _NC * _NS
_L = 16
_ROW = 128


def _cdiv(a, b):
    return (a + b - 1) // b


def kernel(x, edge_index, W, b):
    N, D = x.shape
    E = edge_index.shape[1]

    G = _cdiv(_cdiv(E, _NW), _ROW)
    EPW = G * _ROW
    NPT = _cdiv(N + 1, _NS * _L) * _L
    N_PAD = NPT * _NS
    R = N_PAD // _ROW

    src = edge_index[0].astype(jnp.int32)
    dst = edge_index[1].astype(jnp.int32)
    pad = _NW * EPW - E
    src_p = jnp.concatenate([src, jnp.zeros((pad,), jnp.int32)])
    dst_p = jnp.concatenate([dst, jnp.full((pad,), N, jnp.int32)])
    src3 = src_p.reshape(_NC, _NS, G, _ROW)
    dst3 = dst_p.reshape(_NC, _NS, G, _ROW)

    mesh = plsc.VectorSubcoreMesh(core_axis_name="c", subcore_axis_name="s")

    @functools.partial(
        pl.kernel,
        out_type=jax.ShapeDtypeStruct((_NC, N_PAD), jnp.float32),
        mesh=mesh,
        scratch_types=[
            pltpu.VMEM((G, _ROW), jnp.int32),
            pltpu.VMEM((_ROW,), jnp.float32),
            pltpu.VMEM((NPT,), jnp.float32),
            pltpu.VMEM_SHARED((N_PAD,), jnp.float32),
            pltpu.SemaphoreType.DMA,
        ],
    )
    def _deg(dst_hbm, cnt_hbm, idx_v, ones_v, tmp_v, acc_s, sem):
        c = lax.axis_index("c")
        s = lax.axis_index("s")
        cp = pltpu.async_copy(dst_hbm.at[c, s], idx_v, sem)
        for i in range(_ROW // _L):
            ones_v[pl.ds(i * _L, _L)] = jnp.full((_L,), 1.0, jnp.float32)

        def fz(i, carry):
            tmp_v[pl.ds(i * _L, _L)] = jnp.zeros((_L,), jnp.float32)
            return carry

        lax.fori_loop(0, NPT // _L, fz, 0)
        pltpu.sync_copy(tmp_v, acc_s.at[pl.ds(s * NPT, NPT)])
        cp.wait()
        plsc.subcore_barrier()
        cps = [
            pltpu.async_copy(ones_v, acc_s.at[idx_v.at[j]], sem, add=True)
            for j in range(G)
        ]
        for cp2 in cps:
            cp2.wait()
        plsc.subcore_barrier()
        pltpu.sync_copy(acc_s.at[pl.ds(s * NPT, NPT)], tmp_v)
        pltpu.sync_copy(tmp_v, cnt_hbm.at[c, pl.ds(s * NPT, NPT)])

    cnt = _deg(dst3)

    x_pad = jnp.concatenate(
        [x, jnp.zeros((N_PAD - N, D), jnp.float32)], axis=0)
    cnt3 = cnt.reshape(_NC, R, _ROW)

    def _tc_pre(x_ref, w_ref, cnt_ref, dis_ref, g_ref):
        h = jnp.dot(x_ref[...], w_ref[...],
                    preferred_element_type=jnp.float32)
        deg = cnt_ref[0] + cnt_ref[1] + 1.0
        dis = lax.rsqrt(deg)
        g_ref[...] = h.reshape(R, _ROW) * dis
        dis_ref[...] = dis

    dis2, g2 = pl.pallas_call(
        _tc_pre,
        out_shape=(
            jax.ShapeDtypeStruct((R, _ROW), jnp.float32),
            jax.ShapeDtypeStruct((R, _ROW), jnp.float32),
        ),
    )(x_pad, W, cnt3)
    g_flat = g2.reshape(N_PAD)

    @functools.partial(
        pl.kernel,
        out_type=jax.ShapeDtypeStruct((_NC, N_PAD), jnp.float32),
        mesh=mesh,
        scratch_types=[
            pltpu.VMEM((G, _ROW), jnp.int32),
            pltpu.VMEM((G, _ROW), jnp.int32),
            pltpu.VMEM((G, _ROW), jnp.float32),
            pltpu.VMEM((NPT,), jnp.float32),
            pltpu.VMEM_SHARED((N_PAD,), jnp.float32),
            pltpu.VMEM_SHARED((N_PAD,), jnp.float32),
            pltpu.SemaphoreType.DMA,
        ],
    )
    def _agg(src_hbm, dst_hbm, g_hbm, q_hbm,
             sidx_v, didx_v, vals_v, tmp_v, acc_s, g_s, sem):
        c = lax.axis_index("c")
        s = lax.axis_index("s")
        cp1 = pltpu.async_copy(src_hbm.at[c, s], sidx_v, sem)
        cp2 = pltpu.async_copy(dst_hbm.at[c, s], didx_v, sem)

        def fz(i, carry):
            tmp_v[pl.ds(i * _L, _L)] = jnp.zeros((_L,), jnp.float32)
            return carry

        lax.fori_loop(0, NPT // _L, fz, 0)
        pltpu.sync_copy(tmp_v, acc_s.at[pl.ds(s * NPT, NPT)])
        pltpu.sync_copy(g_hbm.at[pl.ds(s * NPT, NPT)], tmp_v)
        pltpu.sync_copy(tmp_v, g_s.at[pl.ds(s * NPT, NPT)])
        cp1.wait()
        cp2.wait()
        plsc.subcore_barrier()
        gcps = [
            pltpu.async_copy(g_s.at[sidx_v.at[j]], vals_v.at[j], sem)
            for j in range(G)
        ]
        for cpg in gcps:
            cpg.wait()
        cps = [
            pltpu.async_copy(vals_v.at[j], acc_s.at[didx_v.at[j]], sem,
                             add=True)
            for j in range(G)
        ]
        for cp3 in cps:
            cp3.wait()
        plsc.subcore_barrier()
        pltpu.sync_copy(acc_s.at[pl.ds(s * NPT, NPT)], tmp_v)
        pltpu.sync_copy(tmp_v, q_hbm.at[c, pl.ds(s * NPT, NPT)])

    q = _agg(src3, dst3, g_flat)
    q3 = q.reshape(_NC, R, _ROW)

    def _tc_post(dis_ref, g_ref, q_ref, b_ref, out_ref):
        tot = g_ref[...] + q_ref[0] + q_ref[1]
        out_ref[...] = dis_ref[...] * tot + b_ref[0, 0]

    out2 = pl.pallas_call(
        _tc_post,
        out_shape=jax.ShapeDtypeStruct((R, _ROW), jnp.float32),
    )(dis2, g2, q3, b.reshape(1, 1))

    return out2.reshape(N_PAD)[:N, None]


# speedup vs baseline: 107.7906x; 1.0006x over previous
import functools

import jax
import jax.numpy as jnp
from jax import lax
from jax.experimental import pallas as pl
from jax.experimental.pallas import tpu as pltpu
from jax.experimental.pallas import tpu_sc as plsc

_NC = 2
_NS = 16
_NW = _NC * _NS
_L = 16
_ROW = 128


def _cdiv(a, b):
    return (a + b - 1) // b


def kernel(x, edge_index, W, b):
    N, D = x.shape
    E = edge_index.shape[1]

    EPW = _cdiv(_cdiv(E, _NW), _ROW) * _ROW
    NPT = _cdiv(N + 1, _NS * _L) * _L
    N_PAD = NPT * _NS
    R = N_PAD // _ROW

    src = edge_index[0].astype(jnp.int32)
    dst = edge_index[1].astype(jnp.int32)
    pad = _NW * EPW - E
    src_p = jnp.concatenate([src, jnp.zeros((pad,), jnp.int32)])
    dst_p = jnp.concatenate([dst, jnp.full((pad,), N, jnp.int32)])
    src3 = src_p.reshape(_NC, _NS, EPW)
    dst3 = dst_p.reshape(_NC, _NS, EPW)

    mesh = plsc.VectorSubcoreMesh(core_axis_name="c", subcore_axis_name="s")

    @functools.partial(
        pl.kernel,
        out_type=jax.ShapeDtypeStruct((_NC, N_PAD), jnp.float32),
        mesh=mesh,
        scratch_types=[
            pltpu.VMEM((EPW,), jnp.int32),
            pltpu.VMEM((EPW,), jnp.float32),
            pltpu.VMEM((NPT,), jnp.float32),
            pltpu.VMEM_SHARED((N_PAD,), jnp.float32),
            pltpu.SemaphoreType.DMA,
        ],
    )
    def _deg(dst_hbm, cnt_hbm, idx_v, ones_v, tmp_v, acc_s, sem):
        c = lax.axis_index("c")
        s = lax.axis_index("s")
        cp = pltpu.async_copy(dst_hbm.at[c, s], idx_v, sem)

        def fo(i, carry):
            ones_v[pl.ds(i * _L, _L)] = jnp.full((_L,), 1.0, jnp.float32)
            return carry

        lax.fori_loop(0, EPW // _L, fo, 0)

        def fz(i, carry):
            tmp_v[pl.ds(i * _L, _L)] = jnp.zeros((_L,), jnp.float32)
            return carry

        lax.fori_loop(0, NPT // _L, fz, 0)
        pltpu.sync_copy(tmp_v, acc_s.at[pl.ds(s * NPT, NPT)])
        cp.wait()
        plsc.subcore_barrier()
        pltpu.sync_copy(ones_v, acc_s.at[idx_v], add=True)
        plsc.subcore_barrier()
        pltpu.sync_copy(acc_s.at[pl.ds(s * NPT, NPT)], tmp_v)
        pltpu.sync_copy(tmp_v, cnt_hbm.at[c, pl.ds(s * NPT, NPT)])

    cnt = _deg(dst3)

    x_pad = jnp.concatenate(
        [x, jnp.zeros((N_PAD - N, D), jnp.float32)], axis=0)
    cnt3 = cnt.reshape(_NC, R, _ROW)

    def _tc_pre(x_ref, w_ref, cnt_ref, dis_ref, g_ref):
        h = jnp.dot(x_ref[...], w_ref[...],
                    preferred_element_type=jnp.float32)
        deg = cnt_ref[0] + cnt_ref[1] + 1.0
        dis = lax.rsqrt(deg)
        g_ref[...] = h.reshape(R, _ROW) * dis
        dis_ref[...] = dis

    dis2, g2 = pl.pallas_call(
        _tc_pre,
        out_shape=(
            jax.ShapeDtypeStruct((R, _ROW), jnp.float32),
            jax.ShapeDtypeStruct((R, _ROW), jnp.float32),
        ),
    )(x_pad, W, cnt3)
    g_flat = g2.reshape(N_PAD)

    @functools.partial(
        pl.kernel,
        out_type=jax.ShapeDtypeStruct((_NC, N_PAD), jnp.float32),
        mesh=mesh,
        scratch_types=[
            pltpu.VMEM((EPW,), jnp.int32),
            pltpu.VMEM((EPW,), jnp.int32),
            pltpu.VMEM((EPW,), jnp.float32),
            pltpu.VMEM((NPT,), jnp.float32),
            pltpu.VMEM_SHARED((N_PAD,), jnp.float32),
            pltpu.VMEM_SHARED((N_PAD,), jnp.float32),
            pltpu.SemaphoreType.DMA,
        ],
    )
    def _agg(src_hbm, dst_hbm, g_hbm, q_hbm,
             sidx_v, didx_v, vals_v, tmp_v, acc_s, g_s, sem):
        c = lax.axis_index("c")
        s = lax.axis_index("s")
        cp1 = pltpu.async_copy(src_hbm.at[c, s], sidx_v, sem)
        cp2 = pltpu.async_copy(dst_hbm.at[c, s], didx_v, sem)

        def fz(i, carry):
            tmp_v[pl.ds(i * _L, _L)] = jnp.zeros((_L,), jnp.float32)
            return carry

        lax.fori_loop(0, NPT // _L, fz, 0)
        pltpu.sync_copy(tmp_v, acc_s.at[pl.ds(s * NPT, NPT)])
        pltpu.sync_copy(g_hbm.at[pl.ds(s * NPT, NPT)], tmp_v)
        pltpu.sync_copy(tmp_v, g_s.at[pl.ds(s * NPT, NPT)])
        cp1.wait()
        cp2.wait()
        plsc.subcore_barrier()
        pltpu.sync_copy(g_s.at[sidx_v], vals_v)
        pltpu.sync_copy(vals_v, acc_s.at[didx_v], add=True)
        plsc.subcore_barrier()
        pltpu.sync_copy(acc_s.at[pl.ds(s * NPT, NPT)], tmp_v)
        pltpu.sync_copy(tmp_v, q_hbm.at[c, pl.ds(s * NPT, NPT)])

    q = _agg(src3, dst3, g_flat)
    q3 = q.reshape(_NC, R, _ROW)

    def _tc_post(dis_ref, g_ref, q_ref, b_ref, out_ref):
        tot = g_ref[...] + q_ref[0] + q_ref[1]
        out_ref[...] = dis_ref[...] * tot + b_ref[0, 0]

    out2 = pl.pallas_call(
        _tc_post,
        out_shape=jax.ShapeDtypeStruct((R, _ROW), jnp.float32),
    )(dis2, g2, q3, b.reshape(1, 1))

    return out2.reshape(N_PAD)[:N, None]
